# Initial kernel scaffold; baseline (speedup 1.0000x reference)
import functools

import jax
import jax.numpy as jnp
from jax import lax
from jax.experimental import pallas as pl
from jax.experimental.pallas import tpu as pltpu
from jax.experimental.pallas import tpu_sc as plsc

_TB = 512
_NW = 32
_CH = 128
_CNTW = 16


def _assign_body(z_ref, cb_ref, csq_ref, assign_ref, dc_ref):
    pid = pl.program_id(0)
    zb = z_ref[...]
    cb = cb_ref[...]
    K = cb.shape[0]
    dots = lax.dot_general(zb, cb, (((1,), (1,)), ((), ())),
                           preferred_element_type=jnp.float32)
    zsq = jnp.sum(zb * zb, axis=1, keepdims=True)
    d2 = zsq - 2.0 * dots + csq_ref[...]
    m = jnp.min(d2, axis=1, keepdims=True)
    lanes = lax.broadcasted_iota(jnp.int32, d2.shape, 1)
    idx = jnp.min(jnp.where(d2 == m, lanes, K), axis=1, keepdims=True)
    assign_ref[...] = idx

    @pl.when(pid == 0)
    def _():
        dc_ref[0, 0] = 0.0

    dc_ref[0, 0] += jnp.sum(jnp.sqrt(m + 1e-12))


def _tc_assign(flat, codebook, csq):
    T, d = flat.shape
    K = codebook.shape[0]
    nblk = T // _TB
    return pl.pallas_call(
        _assign_body,
        grid=(nblk,),
        in_specs=[
            pl.BlockSpec((_TB, d), lambda i: (i, 0)),
            pl.BlockSpec((K, d), lambda i: (0, 0)),
            pl.BlockSpec((1, K), lambda i: (0, 0)),
        ],
        out_specs=[
            pl.BlockSpec((_TB, 1), lambda i: (i, 0)),
            pl.BlockSpec((1, 1), lambda i: (0, 0)),
        ],
        out_shape=[
            jax.ShapeDtypeStruct((T, 1), jnp.int32),
            jax.ShapeDtypeStruct((1, 1), jnp.float32),
        ],
    )(flat, codebook, csq)


def _make_sc_kernel(T, K, d):
    b_per_w = T // _NW
    nchunk = b_per_w // _CH
    mesh = plsc.VectorSubcoreMesh(core_axis_name="c", subcore_axis_name="s")

    @functools.partial(
        pl.kernel,
        mesh=mesh,
        out_type=[
            jax.ShapeDtypeStruct((T, d), jnp.float32),
            jax.ShapeDtypeStruct((2, K, d), jnp.float32),
            jax.ShapeDtypeStruct((2, K, _CNTW), jnp.float32),
        ],
        scratch_types=[
            pltpu.VMEM((T // _NW, 64), jnp.float32),
            pltpu.VMEM((T // _NW // _CH, _CH), jnp.int32),
            pltpu.VMEM((_CH, 64), jnp.float32),
            pltpu.VMEM((_CH, _CNTW), jnp.float32),
            pltpu.VMEM_SHARED((K, 64), jnp.float32),
            pltpu.VMEM_SHARED((K, _CNTW), jnp.float32),
            pltpu.SemaphoreType.DMA,
        ],
    )
    def sc_kernel(cb_hbm, z_hbm, idx_hbm, quant_hbm, sums_hbm, cnts_hbm,
                  zbuf, idxbuf, qbuf, onebuf, s_sum, s_cnt, sem):
        cid = lax.axis_index("c")
        sid = lax.axis_index("s")
        wid = sid * 2 + cid
        base = wid * b_per_w

        zeros16 = jnp.zeros((16,), jnp.float32)
        ones16 = jnp.ones((16,), jnp.float32)

        @pl.loop(0, 64)
        def _(r):
            @pl.loop(0, d // 16)
            def _(c):
                zbuf[r, pl.ds(c * 16, 16)] = zeros16

        @pl.loop(0, _CH)
        def _(r):
            onebuf[r, pl.ds(0, _CNTW)] = zeros16

        rows_per_sub = K // 16
        pltpu.sync_copy(zbuf.at[pl.ds(0, rows_per_sub)],
                        s_sum.at[pl.ds(sid * rows_per_sub, rows_per_sub)])
        pltpu.sync_copy(onebuf.at[pl.ds(0, rows_per_sub // 2)],
                        s_cnt.at[pl.ds(sid * (rows_per_sub // 2),
                                       rows_per_sub // 2)])

        @pl.loop(0, _CH)
        def _(r):
            onebuf[r, pl.ds(0, _CNTW)] = ones16

        pltpu.sync_copy(z_hbm.at[pl.ds(base, b_per_w)], zbuf)
        pltpu.sync_copy(idx_hbm.at[pl.ds(wid * nchunk, nchunk)], idxbuf)

        plsc.subcore_barrier()

        @pl.loop(0, nchunk)
        def _(c):
            idx_row = idxbuf.at[c]
            pltpu.async_copy(cb_hbm.at[idx_row], qbuf, sem).wait()
            pltpu.sync_copy(qbuf, quant_hbm.at[pl.ds(base + c * _CH, _CH)])
            pltpu.sync_copy(zbuf.at[pl.ds(c * _CH, _CH)],
                            s_sum.at[idx_row], add=True)
            pltpu.sync_copy(onebuf, s_cnt.at[idx_row], add=True)

        plsc.subcore_barrier()

        @pl.when(sid == 0)
        def _():
            pltpu.sync_copy(s_sum, sums_hbm.at[cid])
            pltpu.sync_copy(s_cnt, cnts_hbm.at[cid])

    return sc_kernel


def _finish_body(T, s_ref, c_ref, cb_ref, dc_ref, loss_ref):
    sums = s_ref[0] + s_ref[1]
    cnt = c_ref[0, :, 0:1] + c_ref[1, :, 0:1]
    means = sums / jnp.maximum(cnt, 1.0)
    diff = cb_ref[...] - means
    normsq = jnp.sum(diff * diff, axis=1, keepdims=True)
    mask = (cnt > 0.0).astype(jnp.float32)
    nc_num = jnp.sum(jnp.sqrt(normsq + 1e-12) * mask)
    nc_den = jnp.maximum(jnp.sum(mask), 1.0)
    loss_ref[0, 0] = nc_num / nc_den + dc_ref[0, 0] / T


def _tc_finish(T, sums2, cnts2, codebook, dc_sum):
    K, d = codebook.shape
    return pl.pallas_call(
        functools.partial(_finish_body, T),
        in_specs=[
            pl.BlockSpec((2, K, d), lambda: (0, 0, 0)),
            pl.BlockSpec((2, K, _CNTW), lambda: (0, 0, 0)),
            pl.BlockSpec((K, d), lambda: (0, 0)),
            pl.BlockSpec((1, 1), lambda: (0, 0)),
        ],
        out_specs=pl.BlockSpec((1, 1), lambda: (0, 0)),
        out_shape=jax.ShapeDtypeStruct((1, 1), jnp.float32),
    )(sums2, cnts2, codebook, dc_sum)


def kernel(z, codebook):
    B, N, d = z.shape
    T = B * N
    K = codebook.shape[0]
    flat = z.reshape(T, d)
    csq = jnp.sum(codebook * codebook, axis=1)[None, :]

    assign2d, dc_sum = _tc_assign(flat, codebook, csq)
    assign = assign2d.reshape(T)

    sc = _make_sc_kernel(T, K, d)
    quant_flat, sums2, cnts2 = sc(codebook, flat,
                                  assign.reshape(T // _CH, _CH))

    loss = _tc_finish(T, sums2, cnts2, codebook, dc_sum)
    return loss.reshape(()), quant_flat.reshape(z.shape), assign


# speedup vs baseline: 2.3479x; 2.3479x over previous
import functools

import jax
import jax.numpy as jnp
from jax import lax
from jax.experimental import pallas as pl
from jax.experimental.pallas import tpu as pltpu
from jax.experimental.pallas import tpu_sc as plsc

_TB = 512
_NW = 32
_CH = 128
_CNTW = 16


def _assign_body(z_ref, cb_ref, csq_ref, assign_ref, dc_ref):
    pid = pl.program_id(0)
    zb = z_ref[...]
    cb = cb_ref[...]
    K = cb.shape[0]
    dots = lax.dot_general(zb, cb, (((1,), (1,)), ((), ())),
                           preferred_element_type=jnp.float32)
    zsq = jnp.sum(zb * zb, axis=1, keepdims=True)
    d2 = zsq - 2.0 * dots + csq_ref[...]
    m = jnp.min(d2, axis=1, keepdims=True)
    lanes = lax.broadcasted_iota(jnp.int32, d2.shape, 1)
    idx = jnp.min(jnp.where(d2 == m, lanes, K), axis=1, keepdims=True)
    assign_ref[...] = idx

    @pl.when(pid == 0)
    def _():
        dc_ref[...] = jnp.zeros((1, 1), jnp.float32)

    dc_ref[...] += jnp.sum(jnp.sqrt(m + 1e-12)).reshape(1, 1)


def _tc_assign(flat, codebook, csq):
    T, d = flat.shape
    K = codebook.shape[0]
    nblk = T // _TB
    return pl.pallas_call(
        _assign_body,
        grid=(nblk,),
        in_specs=[
            pl.BlockSpec((_TB, d), lambda i: (i, 0)),
            pl.BlockSpec((K, d), lambda i: (0, 0)),
            pl.BlockSpec((1, K), lambda i: (0, 0)),
        ],
        out_specs=[
            pl.BlockSpec((_TB, 1), lambda i: (i, 0)),
            pl.BlockSpec((1, 1), lambda i: (0, 0)),
        ],
        out_shape=[
            jax.ShapeDtypeStruct((T, 1), jnp.int32),
            jax.ShapeDtypeStruct((1, 1), jnp.float32),
        ],
    )(flat, codebook, csq)


def _make_sc_kernel(T, K, d):
    b_per_w = T // _NW
    nchunk = b_per_w // _CH
    mesh = plsc.VectorSubcoreMesh(core_axis_name="c", subcore_axis_name="s")

    @functools.partial(
        pl.kernel,
        mesh=mesh,
        compiler_params=pltpu.CompilerParams(use_tc_tiling_on_sc=False),
        out_type=[
            jax.ShapeDtypeStruct((T, d), jnp.float32),
            jax.ShapeDtypeStruct((2, K, d), jnp.float32),
            jax.ShapeDtypeStruct((2, K, _CNTW), jnp.float32),
        ],
        scratch_types=[
            pltpu.VMEM((T // _NW, 64), jnp.float32),
            pltpu.VMEM((T // _NW // _CH, _CH), jnp.int32),
            pltpu.VMEM((_CH, 64), jnp.float32),
            pltpu.VMEM((_CH, _CNTW), jnp.float32),
            pltpu.VMEM_SHARED((K, 64), jnp.float32),
            pltpu.VMEM_SHARED((K, _CNTW), jnp.float32),
            pltpu.SemaphoreType.DMA,
        ],
    )
    def sc_kernel(cb_hbm, z_hbm, idx_hbm, quant_hbm, sums_hbm, cnts_hbm,
                  zbuf, idxbuf, qbuf, onebuf, s_sum, s_cnt, sem):
        cid = lax.axis_index("c")
        sid = lax.axis_index("s")
        wid = sid * 2 + cid
        base = wid * b_per_w

        zeros16 = jnp.zeros((16,), jnp.float32)
        ones16 = jnp.ones((16,), jnp.float32)

        @pl.loop(0, 64)
        def _(r):
            @pl.loop(0, d // 16)
            def _(c):
                zbuf[r, pl.ds(c * 16, 16)] = zeros16

        @pl.loop(0, _CH)
        def _(r):
            onebuf[r, pl.ds(0, _CNTW)] = zeros16

        rows_per_sub = K // 16
        pltpu.sync_copy(zbuf.at[pl.ds(0, rows_per_sub)],
                        s_sum.at[pl.ds(sid * rows_per_sub, rows_per_sub)])
        pltpu.sync_copy(onebuf.at[pl.ds(0, rows_per_sub)],
                        s_cnt.at[pl.ds(sid * rows_per_sub, rows_per_sub)])

        @pl.loop(0, _CH)
        def _(r):
            onebuf[r, pl.ds(0, _CNTW)] = ones16

        pltpu.sync_copy(z_hbm.at[pl.ds(base, b_per_w)], zbuf)
        pltpu.sync_copy(idx_hbm.at[wid], idxbuf)

        plsc.subcore_barrier()

        @pl.loop(0, nchunk)
        def _(c):
            idx_row = idxbuf.at[c]
            pltpu.async_copy(cb_hbm.at[idx_row], qbuf, sem).wait()
            pltpu.sync_copy(qbuf, quant_hbm.at[pl.ds(base + c * _CH, _CH)])
            pltpu.sync_copy(zbuf.at[pl.ds(c * _CH, _CH)],
                            s_sum.at[idx_row], add=True)
            pltpu.sync_copy(onebuf, s_cnt.at[idx_row], add=True)

        plsc.subcore_barrier()

        @pl.when(sid == 0)
        def _():
            pltpu.sync_copy(s_sum, sums_hbm.at[cid])
            pltpu.sync_copy(s_cnt, cnts_hbm.at[cid])

    return sc_kernel


def _finish_body(T, s_ref, c_ref, cb_ref, dc_ref, loss_ref):
    sums = s_ref[0] + s_ref[1]
    cnt = c_ref[0, :, 0:1] + c_ref[1, :, 0:1]
    means = sums / jnp.maximum(cnt, 1.0)
    diff = cb_ref[...] - means
    normsq = jnp.sum(diff * diff, axis=1, keepdims=True)
    mask = (cnt > 0.0).astype(jnp.float32)
    nc_num = jnp.sum(jnp.sqrt(normsq + 1e-12) * mask)
    nc_den = jnp.maximum(jnp.sum(mask), 1.0)
    loss_ref[...] = (nc_num / nc_den + dc_ref[...][0, 0] / T).reshape(1, 1)


def _tc_finish(T, sums2, cnts2, codebook, dc_sum):
    K, d = codebook.shape
    return pl.pallas_call(
        functools.partial(_finish_body, T),
        in_specs=[
            pl.BlockSpec((2, K, d), lambda: (0, 0, 0)),
            pl.BlockSpec((2, K, _CNTW), lambda: (0, 0, 0)),
            pl.BlockSpec((K, d), lambda: (0, 0)),
            pl.BlockSpec((1, 1), lambda: (0, 0)),
        ],
        out_specs=pl.BlockSpec((1, 1), lambda: (0, 0)),
        out_shape=jax.ShapeDtypeStruct((1, 1), jnp.float32),
    )(sums2, cnts2, codebook, dc_sum)


def kernel(z, codebook):
    B, N, d = z.shape
    T = B * N
    K = codebook.shape[0]
    flat = z.reshape(T, d)
    csq = jnp.sum(codebook * codebook, axis=1)[None, :]

    assign2d, dc_sum = _tc_assign(flat, codebook, csq)
    assign = assign2d.reshape(T)

    sc = _make_sc_kernel(T, K, d)
    quant_flat, sums2, cnts2 = sc(codebook, flat,
                                  assign.reshape(_NW, T // _NW // _CH, _CH))

    loss = _tc_finish(T, sums2, cnts2, codebook, dc_sum)
    return loss.reshape(()), quant_flat.reshape(z.shape), assign


# speedup vs baseline: 2.9581x; 1.2599x over previous
import functools

import jax
import jax.numpy as jnp
from jax import lax
from jax.experimental import pallas as pl
from jax.experimental.pallas import tpu as pltpu
from jax.experimental.pallas import tpu_sc as plsc

_TB = 1024
_NW = 32
_CH = 128
_W = 128
_NBUF = 3


def _assign_body(z_ref, cb_ref, csq_ref, lanes_ref, assign_ref, dc_ref,
                 zaug_ref):
    pid = pl.program_id(0)
    zb = z_ref[...]
    cb2 = cb_ref[...]
    K = cb2.shape[0]
    dots2 = lax.dot_general(zb, cb2, (((1,), (1,)), ((), ())),
                            preferred_element_type=jnp.float32)
    zsq = jnp.sum(zb * zb, axis=1, keepdims=True)
    d2 = zsq - dots2 + csq_ref[...]
    m = jnp.min(d2, axis=1, keepdims=True)
    idxf = jnp.min(jnp.where(d2 == m, lanes_ref[...], float(K)),
                   axis=1, keepdims=True)
    assign_ref[...] = idxf.astype(jnp.int32).reshape(assign_ref.shape)

    d = zb.shape[1]
    pad = jnp.concatenate(
        [jnp.ones((zb.shape[0], 1), jnp.float32),
         jnp.zeros((zb.shape[0], _W - d - 1), jnp.float32)], axis=1)
    zaug_ref[...] = jnp.concatenate([zb, pad], axis=1)

    @pl.when(pid == 0)
    def _():
        dc_ref[...] = jnp.zeros(dc_ref.shape, jnp.float32)

    dc_ref[...] += jnp.sqrt(m + 1e-12)


def _tc_assign(flat, codebook, csq):
    T, d = flat.shape
    K = codebook.shape[0]
    nblk = T // _TB
    return pl.pallas_call(
        _assign_body,
        grid=(nblk,),
        in_specs=[
            pl.BlockSpec((_TB, d), lambda i: (i, 0)),
            pl.BlockSpec((K, d), lambda i: (0, 0)),
            pl.BlockSpec((1, K), lambda i: (0, 0)),
            pl.BlockSpec((1, K), lambda i: (0, 0)),
        ],
        out_specs=[
            pl.BlockSpec((_TB // _CH, _CH), lambda i: (i, 0)),
            pl.BlockSpec((_TB, 1), lambda i: (0, 0)),
            pl.BlockSpec((_TB, _W), lambda i: (i, 0)),
        ],
        out_shape=[
            jax.ShapeDtypeStruct((T // _CH, _CH), jnp.int32),
            jax.ShapeDtypeStruct((_TB, 1), jnp.float32),
            jax.ShapeDtypeStruct((T, _W), jnp.float32),
        ],
    )(flat, codebook * 2.0, csq,
      jnp.arange(K, dtype=jnp.float32)[None, :])


def _make_sc_kernel(T, K, d):
    b_per_w = T // _NW
    nchunk = b_per_w // _CH
    mesh = plsc.VectorSubcoreMesh(core_axis_name="c", subcore_axis_name="s")

    @functools.partial(
        pl.kernel,
        mesh=mesh,
        out_type=[
            jax.ShapeDtypeStruct((T, _W), jnp.float32),
            jax.ShapeDtypeStruct((2, K, _W), jnp.float32),
        ],
        scratch_types=[
            pltpu.VMEM((_NBUF, _CH, _W), jnp.float32),
            pltpu.VMEM((nchunk, _CH), jnp.int32),
            pltpu.VMEM((_NBUF, _CH, _W), jnp.float32),
            pltpu.VMEM_SHARED((K, _W), jnp.float32),
            pltpu.SemaphoreType.DMA,
            pltpu.SemaphoreType.DMA,
            pltpu.SemaphoreType.DMA,
            pltpu.SemaphoreType.DMA,
        ],
    )
    def sc_kernel(cb_hbm, z_hbm, idx_hbm, quant_hbm, acc_hbm,
                  zbuf, idxbuf, qbuf, s_acc, sem_z, sem_g, sem_q, sem_s):
        cid = lax.axis_index("c")
        sid = lax.axis_index("s")
        wid = sid * 2 + cid

        zeros16 = jnp.zeros((16,), jnp.float32)

        rows_per_sub = K // 16
        @pl.loop(0, rows_per_sub)
        def _(r):
            @pl.loop(0, _W // 16)
            def _(c):
                zbuf[0, r, pl.ds(c * 16, 16)] = zeros16

        pltpu.sync_copy(zbuf.at[0, pl.ds(0, rows_per_sub)],
                        s_acc.at[pl.ds(sid * rows_per_sub, rows_per_sub)])

        hi = [pltpu.async_copy(idx_hbm.at[wid + _NW * c], idxbuf.at[c],
                               sem_z) for c in range(nchunk)]
        for h in hi:
            h.wait()

        plsc.subcore_barrier()

        hq = {}
        hs = {}
        for c in range(nchunk):
            if c >= _NBUF:
                hq[c - _NBUF].wait()
                hs[c - _NBUF].wait()
            b = c % _NBUF
            rows = pl.ds((wid + _NW * c) * _CH, _CH)
            idx_row = idxbuf.at[c]
            hz = pltpu.async_copy(z_hbm.at[rows], zbuf.at[b], sem_z)
            hg = pltpu.async_copy(cb_hbm.at[idx_row], qbuf.at[b], sem_g)
            hg.wait()
            hq[c] = pltpu.async_copy(qbuf.at[b], quant_hbm.at[rows], sem_q)
            hz.wait()
            hs[c] = pltpu.async_copy(zbuf.at[b], s_acc.at[idx_row], sem_s,
                                     add=True)
        for c in range(nchunk - _NBUF, nchunk):
            hq[c].wait()
            hs[c].wait()

        plsc.subcore_barrier()

        @pl.when(sid == 0)
        def _():
            pltpu.sync_copy(s_acc, acc_hbm.at[cid])

    return sc_kernel


def _finish_body(T, d, s_ref, cb_ref, dc_ref, loss_ref):
    acc = s_ref[0] + s_ref[1]
    sums = acc[:, :d]
    cnt = acc[:, d:d + 1]
    means = sums / jnp.maximum(cnt, 1.0)
    diff = cb_ref[...] - means
    normsq = jnp.sum(diff * diff, axis=1, keepdims=True)
    mask = (cnt > 0.0).astype(jnp.float32)
    nc_num = jnp.sum(jnp.sqrt(normsq + 1e-12) * mask)
    nc_den = jnp.maximum(jnp.sum(mask), 1.0)
    loss_ref[...] = (nc_num / nc_den + jnp.sum(dc_ref[...]) / T).reshape(1, 1)


def _tc_finish(T, acc2, codebook, dc_sum):
    K, d = codebook.shape
    return pl.pallas_call(
        functools.partial(_finish_body, T, d),
        in_specs=[
            pl.BlockSpec((2, K, _W), lambda: (0, 0, 0)),
            pl.BlockSpec((K, d), lambda: (0, 0)),
            pl.BlockSpec((_TB, 1), lambda: (0, 0)),
        ],
        out_specs=pl.BlockSpec((1, 1), lambda: (0, 0)),
        out_shape=jax.ShapeDtypeStruct((1, 1), jnp.float32),
    )(acc2, codebook, dc_sum)


def kernel(z, codebook):
    B, N, d = z.shape
    T = B * N
    K = codebook.shape[0]
    flat = z.reshape(T, d)
    csq = jnp.sum(codebook * codebook, axis=1)[None, :]

    assign2d, dc_sum, z_aug = _tc_assign(flat, codebook, csq)
    assign = assign2d.reshape(T)

    sc = _make_sc_kernel(T, K, d)
    cb_pad = jnp.pad(codebook, ((0, 0), (0, _W - d)))
    quant_pad, acc2 = sc(cb_pad, z_aug, assign2d)

    loss = _tc_finish(T, acc2, codebook, dc_sum)
    return loss.reshape(()), quant_pad[:, :d].reshape(z.shape), assign


# speedup vs baseline: 2.9791x; 1.0071x over previous
import functools

import jax
import jax.numpy as jnp
from jax import lax
from jax.experimental import pallas as pl
from jax.experimental.pallas import tpu as pltpu
from jax.experimental.pallas import tpu_sc as plsc

_TB = 1024
_NW = 32
_CH = 128
_W = 128
_QB = 4
_ZB = 2


def _assign_body(z_ref, cb_ref, csq_ref, lanes_ref, assign_ref, dc_ref,
                 zaug_ref):
    zb = z_ref[...]
    cb2 = cb_ref[...]
    K = cb2.shape[0]
    dots2 = lax.dot_general(zb, cb2, (((1,), (1,)), ((), ())),
                            preferred_element_type=jnp.float32)
    zsq = jnp.sum(zb * zb, axis=1, keepdims=True)
    d2 = zsq - dots2 + csq_ref[...]
    m = jnp.min(d2, axis=1, keepdims=True)
    idxf = jnp.min(jnp.where(d2 == m, lanes_ref[...], float(K)),
                   axis=1, keepdims=True)
    assign_ref[...] = idxf.astype(jnp.int32).reshape(assign_ref.shape)

    d = zb.shape[1]
    pad = jnp.concatenate(
        [jnp.ones((zb.shape[0], 1), jnp.float32),
         jnp.zeros((zb.shape[0], _W - d - 1), jnp.float32)], axis=1)
    zaug_ref[...] = jnp.concatenate([zb, pad], axis=1)

    dc_ref[...] = jnp.sqrt(m + 1e-12).reshape(dc_ref.shape)


def _tc_assign(flat, codebook, csq):
    T, d = flat.shape
    K = codebook.shape[0]
    nblk = T // _TB
    return pl.pallas_call(
        _assign_body,
        grid=(nblk,),
        in_specs=[
            pl.BlockSpec((_TB, d), lambda i: (i, 0)),
            pl.BlockSpec((K, d), lambda i: (0, 0)),
            pl.BlockSpec((1, K), lambda i: (0, 0)),
            pl.BlockSpec((1, K), lambda i: (0, 0)),
        ],
        out_specs=[
            pl.BlockSpec((_TB // _CH, _CH), lambda i: (i, 0)),
            pl.BlockSpec((_TB // _CH, _CH), lambda i: (i, 0)),
            pl.BlockSpec((_TB, _W), lambda i: (i, 0)),
        ],
        out_shape=[
            jax.ShapeDtypeStruct((T // _CH, _CH), jnp.int32),
            jax.ShapeDtypeStruct((T // _CH, _CH), jnp.float32),
            jax.ShapeDtypeStruct((T, _W), jnp.float32),
        ],
        compiler_params=pltpu.CompilerParams(
            dimension_semantics=("parallel",)),
    )(flat, codebook * 2.0, csq,
      jnp.arange(K, dtype=jnp.float32)[None, :])


def _make_sc_kernel(T, K, d):
    b_per_w = T // _NW
    nchunk = b_per_w // _CH
    mesh = plsc.VectorSubcoreMesh(core_axis_name="c", subcore_axis_name="s")

    @functools.partial(
        pl.kernel,
        mesh=mesh,
        out_type=[
            jax.ShapeDtypeStruct((T, _W), jnp.float32),
            jax.ShapeDtypeStruct((2, K, _W), jnp.float32),
        ],
        scratch_types=[
            pltpu.VMEM((_ZB, _CH, _W), jnp.float32),
            pltpu.VMEM((nchunk, _CH), jnp.int32),
            pltpu.VMEM((_QB, _CH, _W), jnp.float32),
            pltpu.VMEM_SHARED((K, _W), jnp.float32),
            pltpu.SemaphoreType.DMA,
            pltpu.SemaphoreType.DMA,
            pltpu.SemaphoreType.DMA,
            pltpu.SemaphoreType.DMA,
        ],
    )
    def sc_kernel(cb_hbm, z_hbm, idx_hbm, quant_hbm, acc_hbm,
                  zbuf, idxbuf, qbuf, s_acc, sem_z, sem_g, sem_q, sem_s):
        cid = lax.axis_index("c")
        sid = lax.axis_index("s")
        wid = sid * 2 + cid

        zeros16 = jnp.zeros((16,), jnp.float32)

        rows_per_sub = K // 16
        @pl.loop(0, rows_per_sub)
        def _(r):
            @pl.loop(0, _W // 16)
            def _(c):
                zbuf[0, r, pl.ds(c * 16, 16)] = zeros16

        pltpu.sync_copy(zbuf.at[0, pl.ds(0, rows_per_sub)],
                        s_acc.at[pl.ds(sid * rows_per_sub, rows_per_sub)])

        hi = [pltpu.async_copy(idx_hbm.at[wid + _NW * c], idxbuf.at[c],
                               sem_z) for c in range(nchunk)]
        for h in hi:
            h.wait()

        plsc.subcore_barrier()

        def rows(c):
            return pl.ds((wid + _NW * c) * _CH, _CH)

        def gather(c):
            return pltpu.async_copy(cb_hbm.at[idxbuf.at[c]],
                                    qbuf.at[c % _QB], sem_g)

        def zload(c):
            return pltpu.async_copy(z_hbm.at[rows(c)], zbuf.at[c % _ZB],
                                    sem_z)

        hg = {0: gather(0)}
        if nchunk > 1:
            hg[1] = gather(1)
        hz = {0: zload(0)}
        hq = {}
        hs = {}
        for c in range(nchunk):
            if c + 2 < nchunk:
                if c - 2 >= 0:
                    hq[c - 2].wait()
                hg[c + 2] = gather(c + 2)
            if c + 1 < nchunk:
                if c - 1 >= 0:
                    hs[c - 1].wait()
                hz[c + 1] = zload(c + 1)
            hg[c].wait()
            hq[c] = pltpu.async_copy(qbuf.at[c % _QB], quant_hbm.at[rows(c)],
                                     sem_q)
            hz[c].wait()
            hs[c] = pltpu.async_copy(zbuf.at[c % _ZB], s_acc.at[idxbuf.at[c]],
                                     sem_s, add=True)
        for c in range(max(0, nchunk - 2), nchunk):
            hq[c].wait()
            hs[c].wait()

        plsc.subcore_barrier()

        @pl.when(sid == 0)
        def _():
            pltpu.sync_copy(s_acc, acc_hbm.at[cid])

    return sc_kernel


def _finish_body(T, d, s_ref, cb_ref, dc_ref, loss_ref):
    acc = s_ref[0] + s_ref[1]
    sums = acc[:, :d]
    cnt = acc[:, d:d + 1]
    means = sums / jnp.maximum(cnt, 1.0)
    diff = cb_ref[...] - means
    normsq = jnp.sum(diff * diff, axis=1, keepdims=True)
    mask = (cnt > 0.0).astype(jnp.float32)
    nc_num = jnp.sum(jnp.sqrt(normsq + 1e-12) * mask)
    nc_den = jnp.maximum(jnp.sum(mask), 1.0)
    loss_ref[...] = (nc_num / nc_den + jnp.sum(dc_ref[...]) / T).reshape(1, 1)


def _tc_finish(T, acc2, codebook, dc_sum):
    K, d = codebook.shape
    return pl.pallas_call(
        functools.partial(_finish_body, T, d),
        in_specs=[
            pl.BlockSpec((2, K, _W), lambda: (0, 0, 0)),
            pl.BlockSpec((K, d), lambda: (0, 0)),
            pl.BlockSpec((T // _CH, _CH), lambda: (0, 0)),
        ],
        out_specs=pl.BlockSpec((1, 1), lambda: (0, 0)),
        out_shape=jax.ShapeDtypeStruct((1, 1), jnp.float32),
    )(acc2, codebook, dc_sum)


def kernel(z, codebook):
    B, N, d = z.shape
    T = B * N
    K = codebook.shape[0]
    flat = z.reshape(T, d)
    csq = jnp.sum(codebook * codebook, axis=1)[None, :]

    assign2d, dc_sum, z_aug = _tc_assign(flat, codebook, csq)
    assign = assign2d.reshape(T)

    sc = _make_sc_kernel(T, K, d)
    cb_pad = jnp.pad(codebook, ((0, 0), (0, _W - d)))
    quant_pad, acc2 = sc(cb_pad, z_aug, assign2d)

    loss = _tc_finish(T, acc2, codebook, dc_sum)
    return loss.reshape(()), quant_pad[:, :d].reshape(z.shape), assign


# speedup vs baseline: 3.9299x; 1.3191x over previous
import functools

import jax
import jax.numpy as jnp
from jax import lax
from jax.experimental import pallas as pl
from jax.experimental.pallas import tpu as pltpu
from jax.experimental.pallas import tpu_sc as plsc

_TB = 1024
_NW = 32
_CH = 128
_W = 128
_QB = 4
_ZB = 2


def _assign_body(z_ref, cb_ref, csq_ref, lanes_ref, assign_ref, dc_ref,
                 zaug_ref):
    zb = z_ref[...]
    cb2 = cb_ref[...]
    K = cb2.shape[0]
    dots2 = lax.dot_general(zb, cb2, (((1,), (1,)), ((), ())),
                            preferred_element_type=jnp.float32)
    zsq = jnp.sum(zb * zb, axis=1, keepdims=True)
    d2 = zsq - dots2 + csq_ref[...]
    m = jnp.min(d2, axis=1, keepdims=True)
    idxf = jnp.min(jnp.where(d2 == m, lanes_ref[...], float(K)),
                   axis=1, keepdims=True)
    assign_ref[...] = idxf.astype(jnp.int32).reshape(assign_ref.shape)

    d = zb.shape[1]
    pad = jnp.concatenate(
        [jnp.ones((zb.shape[0], 1), jnp.float32),
         jnp.zeros((zb.shape[0], _W - d - 1), jnp.float32)], axis=1)
    zaug_ref[...] = jnp.concatenate([zb, pad], axis=1)

    dc_ref[...] = jnp.sqrt(m + 1e-12).reshape(dc_ref.shape)


def _tc_assign(flat, codebook, csq):
    T, d = flat.shape
    K = codebook.shape[0]
    nblk = T // _TB
    return pl.pallas_call(
        _assign_body,
        grid=(nblk,),
        in_specs=[
            pl.BlockSpec((_TB, d), lambda i: (i, 0)),
            pl.BlockSpec((K, d), lambda i: (0, 0)),
            pl.BlockSpec((1, K), lambda i: (0, 0)),
            pl.BlockSpec((1, K), lambda i: (0, 0)),
        ],
        out_specs=[
            pl.BlockSpec((_TB // _CH, _CH), lambda i: (i, 0)),
            pl.BlockSpec((_TB // _CH, _CH), lambda i: (i, 0)),
            pl.BlockSpec((_TB, _W), lambda i: (i, 0)),
        ],
        out_shape=[
            jax.ShapeDtypeStruct((T // _CH, _CH), jnp.int32),
            jax.ShapeDtypeStruct((T // _CH, _CH), jnp.float32),
            jax.ShapeDtypeStruct((T, _W), jnp.float32),
        ],
        compiler_params=pltpu.CompilerParams(
            dimension_semantics=("parallel",)),
    )(flat, codebook * 2.0, csq,
      jnp.arange(K, dtype=jnp.float32)[None, :])


def _make_sc_kernel(T, K, d):
    b_per_w = T // _NW
    nchunk = b_per_w // _CH
    mesh = plsc.VectorSubcoreMesh(core_axis_name="c", subcore_axis_name="s")

    @functools.partial(
        pl.kernel,
        mesh=mesh,
        out_type=[
            jax.ShapeDtypeStruct((T, _W), jnp.float32),
            jax.ShapeDtypeStruct((2, K, _W), jnp.float32),
        ],
        scratch_types=[
            pltpu.VMEM((_ZB, _CH, _W), jnp.float32),
            pltpu.VMEM((nchunk, _CH), jnp.int32),
            pltpu.VMEM((_QB, _CH, _W), jnp.float32),
            pltpu.VMEM_SHARED((K, _W), jnp.float32),
            pltpu.VMEM_SHARED((K, _W), jnp.float32),
            pltpu.SemaphoreType.DMA,
            pltpu.SemaphoreType.DMA,
            pltpu.SemaphoreType.DMA,
            pltpu.SemaphoreType.DMA,
        ],
    )
    def sc_kernel(cb_hbm, z_hbm, idx_hbm, quant_hbm, acc_hbm,
                  zbuf, idxbuf, qbuf, s_acc, s_cb, sem_z, sem_g, sem_q, sem_s):
        cid = lax.axis_index("c")
        sid = lax.axis_index("s")
        wid = sid * 2 + cid

        zeros16 = jnp.zeros((16,), jnp.float32)

        rows_per_sub = K // 16
        @pl.loop(0, rows_per_sub)
        def _(r):
            @pl.loop(0, _W // 16)
            def _(c):
                zbuf[0, r, pl.ds(c * 16, 16)] = zeros16

        pltpu.sync_copy(zbuf.at[0, pl.ds(0, rows_per_sub)],
                        s_acc.at[pl.ds(sid * rows_per_sub, rows_per_sub)])

        pltpu.sync_copy(cb_hbm.at[pl.ds(sid * rows_per_sub, rows_per_sub)],
                        s_cb.at[pl.ds(sid * rows_per_sub, rows_per_sub)])

        hi = [pltpu.async_copy(idx_hbm.at[wid + _NW * c], idxbuf.at[c],
                               sem_z) for c in range(nchunk)]
        for h in hi:
            h.wait()

        plsc.subcore_barrier()

        def rows(c):
            return pl.ds((wid + _NW * c) * _CH, _CH)

        def gather(c):
            return pltpu.async_copy(s_cb.at[idxbuf.at[c]],
                                    qbuf.at[c % _QB], sem_g)

        def zload(c):
            return pltpu.async_copy(z_hbm.at[rows(c)], zbuf.at[c % _ZB],
                                    sem_z)

        hg = {0: gather(0)}
        if nchunk > 1:
            hg[1] = gather(1)
        hz = {0: zload(0)}
        hq = {}
        hs = {}
        for c in range(nchunk):
            if c + 2 < nchunk:
                if c - 2 >= 0:
                    hq[c - 2].wait()
                hg[c + 2] = gather(c + 2)
            if c + 1 < nchunk:
                if c - 1 >= 0:
                    hs[c - 1].wait()
                hz[c + 1] = zload(c + 1)
            hg[c].wait()
            hq[c] = pltpu.async_copy(qbuf.at[c % _QB], quant_hbm.at[rows(c)],
                                     sem_q)
            hz[c].wait()
            hs[c] = pltpu.async_copy(zbuf.at[c % _ZB], s_acc.at[idxbuf.at[c]],
                                     sem_s, add=True)
        for c in range(max(0, nchunk - 2), nchunk):
            hq[c].wait()
            hs[c].wait()

        plsc.subcore_barrier()

        @pl.when(sid == 0)
        def _():
            pltpu.sync_copy(s_acc, acc_hbm.at[cid])

    return sc_kernel


def _finish_body(T, d, s_ref, cb_ref, dc_ref, loss_ref):
    acc = s_ref[0] + s_ref[1]
    sums = acc[:, :d]
    cnt = acc[:, d:d + 1]
    means = sums / jnp.maximum(cnt, 1.0)
    diff = cb_ref[...] - means
    normsq = jnp.sum(diff * diff, axis=1, keepdims=True)
    mask = (cnt > 0.0).astype(jnp.float32)
    nc_num = jnp.sum(jnp.sqrt(normsq + 1e-12) * mask)
    nc_den = jnp.maximum(jnp.sum(mask), 1.0)
    loss_ref[...] = (nc_num / nc_den + jnp.sum(dc_ref[...]) / T).reshape(1, 1)


def _tc_finish(T, acc2, codebook, dc_sum):
    K, d = codebook.shape
    return pl.pallas_call(
        functools.partial(_finish_body, T, d),
        in_specs=[
            pl.BlockSpec((2, K, _W), lambda: (0, 0, 0)),
            pl.BlockSpec((K, d), lambda: (0, 0)),
            pl.BlockSpec((T // _CH, _CH), lambda: (0, 0)),
        ],
        out_specs=pl.BlockSpec((1, 1), lambda: (0, 0)),
        out_shape=jax.ShapeDtypeStruct((1, 1), jnp.float32),
    )(acc2, codebook, dc_sum)


def kernel(z, codebook):
    B, N, d = z.shape
    T = B * N
    K = codebook.shape[0]
    flat = z.reshape(T, d)
    csq = jnp.sum(codebook * codebook, axis=1)[None, :]

    assign2d, dc_sum, z_aug = _tc_assign(flat, codebook, csq)
    assign = assign2d.reshape(T)

    sc = _make_sc_kernel(T, K, d)
    cb_pad = jnp.pad(codebook, ((0, 0), (0, _W - d)))
    quant_pad, acc2 = sc(cb_pad, z_aug, assign2d)

    loss = _tc_finish(T, acc2, codebook, dc_sum)
    return loss.reshape(()), quant_pad[:, :d].reshape(z.shape), assign


# speedup vs baseline: 4.0587x; 1.0328x over previous
import functools

import jax
import jax.numpy as jnp
from jax import lax
from jax.experimental import pallas as pl
from jax.experimental.pallas import tpu as pltpu
from jax.experimental.pallas import tpu_sc as plsc

_TB = 2048
_NW = 32
_CH = 128
_W = 128
_QB = 4
_ZB = 2


def _assign_body(z_ref, cb_ref, csq_ref, lanes_ref, assign_ref, dc_ref,
                 zaug_ref):
    zb = z_ref[...]
    cb2 = cb_ref[...]
    K = cb2.shape[0]
    dots2 = lax.dot_general(zb, cb2, (((1,), (1,)), ((), ())),
                            preferred_element_type=jnp.float32)
    zsq = jnp.sum(zb * zb, axis=1, keepdims=True)
    d2 = zsq - dots2 + csq_ref[...]
    m = jnp.min(d2, axis=1, keepdims=True)
    idxf = jnp.min(jnp.where(d2 == m, lanes_ref[...], float(K)),
                   axis=1, keepdims=True)
    assign_ref[...] = idxf.astype(jnp.int32).reshape(assign_ref.shape)

    d = zb.shape[1]
    pad = jnp.concatenate(
        [jnp.ones((zb.shape[0], 1), jnp.float32),
         jnp.zeros((zb.shape[0], _W - d - 1), jnp.float32)], axis=1)
    zaug_ref[...] = jnp.concatenate([zb, pad], axis=1)

    dc_ref[...] = jnp.sqrt(m + 1e-12).reshape(dc_ref.shape)


def _tc_assign(flat, codebook, csq):
    T, d = flat.shape
    K = codebook.shape[0]
    nblk = T // _TB
    return pl.pallas_call(
        _assign_body,
        grid=(nblk,),
        in_specs=[
            pl.BlockSpec((_TB, d), lambda i: (i, 0)),
            pl.BlockSpec((K, d), lambda i: (0, 0)),
            pl.BlockSpec((1, K), lambda i: (0, 0)),
            pl.BlockSpec((1, K), lambda i: (0, 0)),
        ],
        out_specs=[
            pl.BlockSpec((_TB // _CH, _CH), lambda i: (i, 0)),
            pl.BlockSpec((_TB // _CH, _CH), lambda i: (i, 0)),
            pl.BlockSpec((_TB, _W), lambda i: (i, 0)),
        ],
        out_shape=[
            jax.ShapeDtypeStruct((T // _CH, _CH), jnp.int32),
            jax.ShapeDtypeStruct((T // _CH, _CH), jnp.float32),
            jax.ShapeDtypeStruct((T, _W), jnp.float32),
        ],
        compiler_params=pltpu.CompilerParams(
            dimension_semantics=("parallel",)),
    )(flat, codebook * 2.0, csq,
      jnp.arange(K, dtype=jnp.float32)[None, :])


def _make_sc_kernel(T, K, d):
    b_per_w = T // _NW
    nchunk = b_per_w // _CH
    mesh = plsc.VectorSubcoreMesh(core_axis_name="c", subcore_axis_name="s")

    @functools.partial(
        pl.kernel,
        mesh=mesh,
        out_type=[
            jax.ShapeDtypeStruct((T, _W), jnp.float32),
            jax.ShapeDtypeStruct((2, K, _W), jnp.float32),
        ],
        scratch_types=[
            pltpu.VMEM((_ZB, _CH, _W), jnp.float32),
            pltpu.VMEM((nchunk, _CH), jnp.int32),
            pltpu.VMEM((_QB, _CH, _W), jnp.float32),
            pltpu.VMEM_SHARED((K, _W), jnp.float32),
            pltpu.VMEM_SHARED((K, _W), jnp.float32),
            pltpu.SemaphoreType.DMA,
            pltpu.SemaphoreType.DMA,
            pltpu.SemaphoreType.DMA,
            pltpu.SemaphoreType.DMA,
        ],
    )
    def sc_kernel(cb_hbm, z_hbm, idx_hbm, quant_hbm, acc_hbm,
                  zbuf, idxbuf, qbuf, s_acc, s_cb, sem_z, sem_g, sem_q, sem_s):
        cid = lax.axis_index("c")
        sid = lax.axis_index("s")
        wid = sid * 2 + cid

        zeros16 = jnp.zeros((16,), jnp.float32)

        rows_per_sub = K // 16
        @pl.loop(0, rows_per_sub)
        def _(r):
            @pl.loop(0, _W // 16)
            def _(c):
                zbuf[0, r, pl.ds(c * 16, 16)] = zeros16

        pltpu.sync_copy(zbuf.at[0, pl.ds(0, rows_per_sub)],
                        s_acc.at[pl.ds(sid * rows_per_sub, rows_per_sub)])

        pltpu.sync_copy(cb_hbm.at[pl.ds(sid * rows_per_sub, rows_per_sub)],
                        s_cb.at[pl.ds(sid * rows_per_sub, rows_per_sub)])

        hi = [pltpu.async_copy(idx_hbm.at[wid + _NW * c], idxbuf.at[c],
                               sem_z) for c in range(nchunk)]
        for h in hi:
            h.wait()

        plsc.subcore_barrier()

        def rows(c):
            return pl.ds((wid + _NW * c) * _CH, _CH)

        def gather(c):
            return pltpu.async_copy(s_cb.at[idxbuf.at[c]],
                                    qbuf.at[c % _QB], sem_g)

        def zload(c):
            return pltpu.async_copy(z_hbm.at[rows(c)], zbuf.at[c % _ZB],
                                    sem_z)

        hg = {0: gather(0)}
        if nchunk > 1:
            hg[1] = gather(1)
        hz = {0: zload(0)}
        hq = {}
        hs = {}
        for c in range(nchunk):
            if c + 2 < nchunk:
                if c - 2 >= 0:
                    hq[c - 2].wait()
                hg[c + 2] = gather(c + 2)
            if c + 1 < nchunk:
                if c - 1 >= 0:
                    hs[c - 1].wait()
                hz[c + 1] = zload(c + 1)
            hg[c].wait()
            hq[c] = pltpu.async_copy(qbuf.at[c % _QB], quant_hbm.at[rows(c)],
                                     sem_q)
            hz[c].wait()
            hs[c] = pltpu.async_copy(zbuf.at[c % _ZB], s_acc.at[idxbuf.at[c]],
                                     sem_s, add=True)
        for c in range(max(0, nchunk - 2), nchunk):
            hq[c].wait()
            hs[c].wait()

        plsc.subcore_barrier()

        @pl.when(sid == 0)
        def _():
            pltpu.sync_copy(s_acc, acc_hbm.at[cid])

    return sc_kernel


def _finish_body(T, d, s_ref, cb_ref, dc_ref, loss_ref):
    acc = s_ref[0] + s_ref[1]
    sums = acc[:, :d]
    cnt = acc[:, d:d + 1]
    means = sums / jnp.maximum(cnt, 1.0)
    diff = cb_ref[...] - means
    normsq = jnp.sum(diff * diff, axis=1, keepdims=True)
    mask = (cnt > 0.0).astype(jnp.float32)
    nc_num = jnp.sum(jnp.sqrt(normsq + 1e-12) * mask)
    nc_den = jnp.maximum(jnp.sum(mask), 1.0)
    loss_ref[...] = (nc_num / nc_den + jnp.sum(dc_ref[...]) / T).reshape(1, 1)


def _tc_finish(T, acc2, codebook, dc_sum):
    K, d = codebook.shape
    return pl.pallas_call(
        functools.partial(_finish_body, T, d),
        in_specs=[
            pl.BlockSpec((2, K, _W), lambda: (0, 0, 0)),
            pl.BlockSpec((K, d), lambda: (0, 0)),
            pl.BlockSpec((T // _CH, _CH), lambda: (0, 0)),
        ],
        out_specs=pl.BlockSpec((1, 1), lambda: (0, 0)),
        out_shape=jax.ShapeDtypeStruct((1, 1), jnp.float32),
    )(acc2, codebook, dc_sum)


def kernel(z, codebook):
    B, N, d = z.shape
    T = B * N
    K = codebook.shape[0]
    flat = z.reshape(T, d)
    csq = jnp.sum(codebook * codebook, axis=1)[None, :]

    assign2d, dc_sum, z_aug = _tc_assign(flat, codebook, csq)
    assign = assign2d.reshape(T)

    sc = _make_sc_kernel(T, K, d)
    cb_pad = jnp.pad(codebook, ((0, 0), (0, _W - d)))
    quant_pad, acc2 = sc(cb_pad, z_aug, assign2d)

    loss = _tc_finish(T, acc2, codebook, dc_sum)
    return loss.reshape(()), quant_pad[:, :d].reshape(z.shape), assign
